# baseline (speedup 1.0000x reference)
import functools

import jax
import jax.numpy as jnp
from jax import lax
from jax.experimental import pallas as pl
from jax.experimental.pallas import tpu as pltpu
from jax.experimental.pallas import tpu_sc as plsc

_N = 128
_M = 100000
_T = 16384
_MT = 4000
_GRID = _M // _MT

_NW = 32
_BPW = _T // _NW
_CROWS = 64
_NCH = _BPW // _CROWS
_NV = _N // 16


_CH = 800
_NCHUNK = _M // _CH
_NBUF = 8


def _lse_body(at_hbm, lse_ref, bufs, sems):
    def _start(c, s):
        pltpu.make_async_copy(
            at_hbm.at[pl.ds(c * _CH, _CH)], bufs.at[s], sems.at[s]
        ).start()

    for s in range(_NBUF):
        _start(s, s)

    def step(c, acc):
        s = c % _NBUF
        pltpu.make_async_copy(
            at_hbm.at[pl.ds(c * _CH, _CH)], bufs.at[s], sems.at[s]
        ).wait()
        acc = acc + jnp.sum(
            jnp.exp(bufs[s]).reshape(_CH // 32, 32, _N), axis=0
        )

        @pl.when(c + _NBUF < _NCHUNK)
        def _next():
            _start(c + _NBUF, s)

        return acc

    acc = lax.fori_loop(
        0, _NCHUNK, step, jnp.zeros((32, _N), jnp.float32)
    )
    lse_ref[...] = jnp.log(jnp.sum(acc, axis=0, keepdims=True))


def _lse_pass(at):
    return pl.pallas_call(
        _lse_body,
        in_specs=[pl.BlockSpec(memory_space=pl.ANY)],
        out_specs=pl.BlockSpec(memory_space=pltpu.VMEM),
        out_shape=jax.ShapeDtypeStruct((1, _N), jnp.float32),
        scratch_shapes=[
            pltpu.VMEM((_NBUF, _CH, _N), jnp.float32),
            pltpu.SemaphoreType.DMA((_NBUF,)),
        ],
    )(at)


@functools.cache
def _make_sc_gather():
    mesh = plsc.VectorSubcoreMesh(core_axis_name="c", subcore_axis_name="s")
    return pl.kernel(
        _sc_gather_body,
        mesh=mesh,
        out_type=jax.ShapeDtypeStruct((_T, _N), jnp.float32),
        scratch_types=[
            pltpu.VMEM((_BPW,), jnp.int32),
            pltpu.VMEM((_BPW, _N), jnp.float32),
            pltpu.VMEM((1, _N), jnp.float32),
            pltpu.SemaphoreType.DMA((_NCH,)),
            pltpu.SemaphoreType.DMA,
        ],
        compiler_params=pltpu.CompilerParams(use_tc_tiling_on_sc=True),
    )


def _sc_gather_body(at_hbm, obs_hbm, lse_hbm, out_hbm, idx_v, rows_v, lse_v,
                    gsems, osem):
    wid = lax.axis_index("s") * 2 + lax.axis_index("c")
    base = wid * _BPW
    pltpu.sync_copy(obs_hbm.at[pl.ds(base, _BPW)], idx_v)
    gathers = [
        pltpu.async_copy(
            at_hbm.at[idx_v.at[pl.ds(j * _CROWS, _CROWS)]],
            rows_v.at[pl.ds(j * _CROWS, _CROWS)],
            gsems.at[j],
        )
        for j in range(_NCH)
    ]
    pltpu.sync_copy(lse_hbm, lse_v)
    lvs = [lse_v[0, pl.ds(16 * j, 16)] for j in range(_NV)]

    outs = []
    for j in range(_NCH):
        gathers[j].wait()

        def body(i, carry, _j=j):
            for k in range(_NV):
                sl = pl.ds(16 * k, 16)
                rows_v[_j * _CROWS + i, sl] = (
                    rows_v[_j * _CROWS + i, sl] - lvs[k]
                )
            return carry

        lax.fori_loop(0, _CROWS, body, 0)
        outs.append(
            pltpu.async_copy(
                rows_v.at[pl.ds(j * _CROWS, _CROWS)],
                out_hbm.at[pl.ds(base + j * _CROWS, _CROWS)],
                osem,
            )
        )
    for c in outs:
        c.wait()


def kernel(obervation_raw, unnormalized_emission_matrix):
    obs = obervation_raw.astype(jnp.int32)
    at = unnormalized_emission_matrix.T
    lse2 = _lse_pass(at)
    return _make_sc_gather()(at, obs, lse2)


# speedup vs baseline: 1.0776x; 1.0776x over previous
import functools

import jax
import jax.numpy as jnp
from jax import lax
from jax.experimental import pallas as pl
from jax.experimental.pallas import tpu as pltpu
from jax.experimental.pallas import tpu_sc as plsc

_N = 128
_M = 100000
_T = 16384
_MT = 4000
_GRID = _M // _MT

_NW = 32
_BPW = _T // _NW
_CROWS = 64
_NCH = _BPW // _CROWS
_NV = _N // 16


_CH = 4000
_NCHUNK = _M // _CH
_NBUF = 5


def _lse_body(at_hbm, lse_ref, bufs, sems):
    def _start(c, s):
        pltpu.make_async_copy(
            at_hbm.at[pl.ds(c * _CH, _CH)], bufs.at[s], sems.at[s]
        ).start()

    for s in range(_NBUF):
        _start(s, s)

    def step(c, acc):
        s = c % _NBUF
        pltpu.make_async_copy(
            at_hbm.at[pl.ds(c * _CH, _CH)], bufs.at[s], sems.at[s]
        ).wait()
        acc = acc + jnp.sum(
            jnp.exp(bufs[s]).reshape(_CH // 32, 32, _N), axis=0
        )

        @pl.when(c + _NBUF < _NCHUNK)
        def _next():
            _start(c + _NBUF, s)

        return acc

    acc = lax.fori_loop(
        0, _NCHUNK, step, jnp.zeros((32, _N), jnp.float32)
    )
    lse_ref[...] = jnp.log(jnp.sum(acc, axis=0, keepdims=True))


def _lse_pass(at):
    return pl.pallas_call(
        _lse_body,
        in_specs=[pl.BlockSpec(memory_space=pl.ANY)],
        out_specs=pl.BlockSpec(memory_space=pltpu.VMEM),
        out_shape=jax.ShapeDtypeStruct((1, _N), jnp.float32),
        scratch_shapes=[
            pltpu.VMEM((_NBUF, _CH, _N), jnp.float32),
            pltpu.SemaphoreType.DMA((_NBUF,)),
        ],
    )(at)


@functools.cache
def _make_sc_gather():
    mesh = plsc.VectorSubcoreMesh(core_axis_name="c", subcore_axis_name="s")
    return pl.kernel(
        _sc_gather_body,
        mesh=mesh,
        out_type=jax.ShapeDtypeStruct((_T, _N), jnp.float32),
        scratch_types=[
            pltpu.VMEM((_BPW,), jnp.int32),
            pltpu.VMEM((_BPW, _N), jnp.float32),
            pltpu.VMEM((1, _N), jnp.float32),
            pltpu.SemaphoreType.DMA((_NCH,)),
            pltpu.SemaphoreType.DMA,
        ],
        compiler_params=pltpu.CompilerParams(use_tc_tiling_on_sc=True),
    )


def _sc_gather_body(at_hbm, obs_hbm, lse_hbm, out_hbm, idx_v, rows_v, lse_v,
                    gsems, osem):
    wid = lax.axis_index("s") * 2 + lax.axis_index("c")
    base = wid * _BPW
    pltpu.sync_copy(obs_hbm.at[pl.ds(base, _BPW)], idx_v)
    gathers = [
        pltpu.async_copy(
            at_hbm.at[idx_v.at[pl.ds(j * _CROWS, _CROWS)]],
            rows_v.at[pl.ds(j * _CROWS, _CROWS)],
            gsems.at[j],
        )
        for j in range(_NCH)
    ]
    pltpu.sync_copy(lse_hbm, lse_v)
    lvs = [lse_v[0, pl.ds(16 * j, 16)] for j in range(_NV)]

    outs = []
    for j in range(_NCH):
        gathers[j].wait()

        def body(i, carry, _j=j):
            for k in range(_NV):
                sl = pl.ds(16 * k, 16)
                rows_v[_j * _CROWS + i, sl] = (
                    rows_v[_j * _CROWS + i, sl] - lvs[k]
                )
            return carry

        lax.fori_loop(0, _CROWS, body, 0)
        outs.append(
            pltpu.async_copy(
                rows_v.at[pl.ds(j * _CROWS, _CROWS)],
                out_hbm.at[pl.ds(base + j * _CROWS, _CROWS)],
                osem,
            )
        )
    for c in outs:
        c.wait()


def kernel(obervation_raw, unnormalized_emission_matrix):
    obs = obervation_raw.astype(jnp.int32)
    at = unnormalized_emission_matrix.T
    lse2 = _lse_pass(at)
    return _make_sc_gather()(at, obs, lse2)


# speedup vs baseline: 1.1125x; 1.0324x over previous
import functools

import jax
import jax.numpy as jnp
from jax import lax
from jax.experimental import pallas as pl
from jax.experimental.pallas import tpu as pltpu
from jax.experimental.pallas import tpu_sc as plsc

_N = 128
_M = 100000
_T = 16384
_MT = 4000
_GRID = _M // _MT

_NW = 32
_BPW = _T // _NW
_CROWS = 128
_NCH = _BPW // _CROWS
_NV = _N // 16


_CH = 4000
_NCHUNK = _M // _CH
_NBUF = 5


def _lse_body(at_hbm, lse_ref, bufs, sems):
    def _start(c, s):
        pltpu.make_async_copy(
            at_hbm.at[pl.ds(c * _CH, _CH)], bufs.at[s], sems.at[s]
        ).start()

    for s in range(_NBUF):
        _start(s, s)

    def step(c, acc):
        s = c % _NBUF
        pltpu.make_async_copy(
            at_hbm.at[pl.ds(c * _CH, _CH)], bufs.at[s], sems.at[s]
        ).wait()
        acc = acc + jnp.sum(
            jnp.exp(bufs[s]).reshape(_CH // 32, 32, _N), axis=0
        )

        @pl.when(c + _NBUF < _NCHUNK)
        def _next():
            _start(c + _NBUF, s)

        return acc

    acc = lax.fori_loop(
        0, _NCHUNK, step, jnp.zeros((32, _N), jnp.float32)
    )
    lse_ref[...] = jnp.log(jnp.sum(acc, axis=0, keepdims=True))


def _lse_pass(at):
    return pl.pallas_call(
        _lse_body,
        in_specs=[pl.BlockSpec(memory_space=pl.ANY)],
        out_specs=pl.BlockSpec(memory_space=pltpu.VMEM),
        out_shape=jax.ShapeDtypeStruct((1, _N), jnp.float32),
        scratch_shapes=[
            pltpu.VMEM((_NBUF, _CH, _N), jnp.float32),
            pltpu.SemaphoreType.DMA((_NBUF,)),
        ],
    )(at)


@functools.cache
def _make_sc_gather():
    mesh = plsc.VectorSubcoreMesh(core_axis_name="c", subcore_axis_name="s")
    return pl.kernel(
        _sc_gather_body,
        mesh=mesh,
        out_type=jax.ShapeDtypeStruct((_T, _N), jnp.float32),
        scratch_types=[
            pltpu.VMEM((_BPW,), jnp.int32),
            pltpu.VMEM((_BPW, _N), jnp.float32),
            pltpu.VMEM((1, _N), jnp.float32),
            pltpu.SemaphoreType.DMA((_NCH,)),
            pltpu.SemaphoreType.DMA,
        ],
        compiler_params=pltpu.CompilerParams(use_tc_tiling_on_sc=True),
    )


def _sc_gather_body(at_hbm, obs_hbm, lse_hbm, out_hbm, idx_v, rows_v, lse_v,
                    gsems, osem):
    wid = lax.axis_index("s") * 2 + lax.axis_index("c")
    base = wid * _BPW
    pltpu.sync_copy(obs_hbm.at[pl.ds(base, _BPW)], idx_v)
    gathers = [
        pltpu.async_copy(
            at_hbm.at[idx_v.at[pl.ds(j * _CROWS, _CROWS)]],
            rows_v.at[pl.ds(j * _CROWS, _CROWS)],
            gsems.at[j],
        )
        for j in range(_NCH)
    ]
    pltpu.sync_copy(lse_hbm, lse_v)
    lvs = [lse_v[0, pl.ds(16 * j, 16)] for j in range(_NV)]

    outs = []
    for j in range(_NCH):
        gathers[j].wait()

        def body(i, carry, _j=j):
            for k in range(_NV):
                sl = pl.ds(16 * k, 16)
                rows_v[_j * _CROWS + i, sl] = (
                    rows_v[_j * _CROWS + i, sl] - lvs[k]
                )
            return carry

        lax.fori_loop(0, _CROWS, body, 0)
        outs.append(
            pltpu.async_copy(
                rows_v.at[pl.ds(j * _CROWS, _CROWS)],
                out_hbm.at[pl.ds(base + j * _CROWS, _CROWS)],
                osem,
            )
        )
    for c in outs:
        c.wait()


def kernel(obervation_raw, unnormalized_emission_matrix):
    obs = obervation_raw.astype(jnp.int32)
    at = unnormalized_emission_matrix.T
    lse2 = _lse_pass(at)
    return _make_sc_gather()(at, obs, lse2)


# speedup vs baseline: 1.1125x; 1.0000x over previous
import functools

import jax
import jax.numpy as jnp
from jax import lax
from jax.experimental import pallas as pl
from jax.experimental.pallas import tpu as pltpu
from jax.experimental.pallas import tpu_sc as plsc

_N = 128
_M = 100000
_T = 16384

_NW = 32
_BPW = _T // _NW
_CROWS = 128
_NCH = _BPW // _CROWS
_NV = _N // 16


_CH = 4000
_NCHUNK = _M // _CH
_NBUF = 5


def _lse_body(at_hbm, lse_ref, bufs, sems):
    def _start(c, s):
        pltpu.make_async_copy(
            at_hbm.at[pl.ds(c * _CH, _CH)], bufs.at[s], sems.at[s]
        ).start()

    for s in range(_NBUF):
        _start(s, s)

    def step(c, acc):
        s = c % _NBUF
        pltpu.make_async_copy(
            at_hbm.at[pl.ds(c * _CH, _CH)], bufs.at[s], sems.at[s]
        ).wait()
        acc = acc + jnp.sum(
            jnp.exp(bufs[s]).reshape(_CH // 32, 32, _N), axis=0
        )

        @pl.when(c + _NBUF < _NCHUNK)
        def _next():
            _start(c + _NBUF, s)

        return acc

    acc = lax.fori_loop(
        0, _NCHUNK, step, jnp.zeros((32, _N), jnp.float32)
    )
    lse_ref[...] = jnp.log(jnp.sum(acc, axis=0, keepdims=True))


def _lse_pass(at):
    return pl.pallas_call(
        _lse_body,
        in_specs=[pl.BlockSpec(memory_space=pl.ANY)],
        out_specs=pl.BlockSpec(memory_space=pltpu.VMEM),
        out_shape=jax.ShapeDtypeStruct((1, _N), jnp.float32),
        scratch_shapes=[
            pltpu.VMEM((_NBUF, _CH, _N), jnp.float32),
            pltpu.SemaphoreType.DMA((_NBUF,)),
        ],
    )(at)


@functools.cache
def _make_sc_gather():
    mesh = plsc.VectorSubcoreMesh(core_axis_name="c", subcore_axis_name="s")
    return pl.kernel(
        _sc_gather_body,
        mesh=mesh,
        out_type=jax.ShapeDtypeStruct((_T, _N), jnp.float32),
        scratch_types=[
            pltpu.VMEM((_BPW,), jnp.int32),
            pltpu.VMEM((_BPW, _N), jnp.float32),
            pltpu.VMEM((1, _N), jnp.float32),
            pltpu.SemaphoreType.DMA((_NCH,)),
            pltpu.SemaphoreType.DMA,
        ],
        compiler_params=pltpu.CompilerParams(use_tc_tiling_on_sc=True),
    )


def _sc_gather_body(at_hbm, obs_hbm, lse_hbm, out_hbm, idx_v, rows_v, lse_v,
                    gsems, osem):
    wid = lax.axis_index("s") * 2 + lax.axis_index("c")
    base = wid * _BPW
    pltpu.sync_copy(obs_hbm.at[pl.ds(base, _BPW)], idx_v)
    gathers = [
        pltpu.async_copy(
            at_hbm.at[idx_v.at[pl.ds(j * _CROWS, _CROWS)]],
            rows_v.at[pl.ds(j * _CROWS, _CROWS)],
            gsems.at[j],
        )
        for j in range(_NCH)
    ]
    pltpu.sync_copy(lse_hbm, lse_v)
    lvs = [lse_v[0, pl.ds(16 * j, 16)] for j in range(_NV)]

    outs = []
    for j in range(_NCH):
        gathers[j].wait()

        def body(i, carry, _j=j):
            for k in range(_NV):
                sl = pl.ds(16 * k, 16)
                rows_v[_j * _CROWS + i, sl] = (
                    rows_v[_j * _CROWS + i, sl] - lvs[k]
                )
            return carry

        lax.fori_loop(0, _CROWS, body, 0)
        outs.append(
            pltpu.async_copy(
                rows_v.at[pl.ds(j * _CROWS, _CROWS)],
                out_hbm.at[pl.ds(base + j * _CROWS, _CROWS)],
                osem,
            )
        )
    for c in outs:
        c.wait()


def kernel(obervation_raw, unnormalized_emission_matrix):
    obs = obervation_raw.astype(jnp.int32)
    at = unnormalized_emission_matrix.T
    lse2 = _lse_pass(at)
    return _make_sc_gather()(at, obs, lse2)
